# baseline (speedup 1.0000x reference)
import jax
import jax.numpy as jnp
from jax import lax
from jax.experimental import pallas as pl

VOCAB = 1000
MAXLEN = 512
WIDTH = VOCAB + MAXLEN


def _body(x_ref, o_ref):
    xv = x_ref[...]
    col = lax.broadcasted_iota(jnp.int32, o_ref.shape, 2)
    pos = lax.broadcasted_iota(jnp.int32, o_ref.shape, 1) + VOCAB
    hit = (col == xv[:, :, None]) | (col == pos)
    o_ref[...] = hit.astype(jnp.float32)


def kernel(x):
    b, s = x.shape
    RB = 16
    return pl.pallas_call(
        _body,
        grid=(b // RB,),
        in_specs=[pl.BlockSpec((RB, s), lambda i: (i, 0))],
        out_specs=pl.BlockSpec((RB, s, WIDTH), lambda i: (i, 0, 0)),
        out_shape=jax.ShapeDtypeStruct((b, s, WIDTH), jnp.float32),
    )(x)
